# baseline (speedup 1.0000x reference)
import functools
import jax
import jax.numpy as jnp
from jax import lax
from jax.experimental import pallas as pl
from jax.experimental.pallas import tpu as pltpu, tpu_sc as plsc

_NC, _NS = 2, 16
_NW = _NC * _NS
_L = 16
_CR = 32
_D = 1024
_SC_BATCH = 1
_TS = 512


def _sc_body(x_hbm, pe_hbm, out_hbm, pe_buf, xb0, xb1, l0, l1, s0, s1,
             *, n_batch, seq):
    wid = lax.axis_index("s") * _NC + lax.axis_index("c")
    slabs = seq // (2 * _CR)
    per0 = (wid % slabs) * (2 * _CR)
    xb = (xb0, xb1)
    lsem = (l0, l1)
    ssem = (s0, s1)
    pending_store = [None, None]

    def row0(p, b):
        return b * seq + per0 + p * _CR

    def start_load(p, b, j):
        if pending_store[j] is not None:
            pending_store[j].wait()
            pending_store[j] = None
        return pltpu.async_copy(x_hbm.at[pl.ds(row0(p, b), _CR), :], xb[j],
                                lsem[j])

    chunks = [(p, b) for p in range(2) for b in range(n_batch)]
    load = start_load(*chunks[0], 0)
    for i, (p, b) in enumerate(chunks):
        j = i % 2
        if b == 0:
            pltpu.sync_copy(pe_hbm.at[pl.ds(per0 + p * _CR, _CR), :], pe_buf)
        load.wait()
        if i + 1 < len(chunks):
            load = start_load(*chunks[i + 1], (i + 1) % 2)

        @plsc.parallel_loop(0, _CR, step=1, unroll=1)
        def _(r):
            for c in range(0, _D, _L):
                plsc.addupdate(xb[j].at[r, pl.ds(c, _L)],
                               pe_buf[r, pl.ds(c, _L)])

        pending_store[j] = pltpu.async_copy(
            xb[j], out_hbm.at[pl.ds(row0(p, b), _CR), :], ssem[j])
    for j in range(2):
        if pending_store[j] is not None:
            pending_store[j].wait()


def _sc_add(x2d, pe_table, k, seq):
    mesh = plsc.VectorSubcoreMesh(core_axis_name="c", subcore_axis_name="s",
                                  num_cores=_NC, num_subcores=_NS)
    return pl.kernel(
        functools.partial(_sc_body, n_batch=k, seq=seq),
        out_type=jax.ShapeDtypeStruct((k * seq, _D), jnp.float32),
        mesh=mesh,
        scratch_types=[
            pltpu.VMEM((_CR, _D), jnp.float32),
            pltpu.VMEM((_CR, _D), jnp.float32),
            pltpu.VMEM((_CR, _D), jnp.float32),
            pltpu.SemaphoreType.DMA,
            pltpu.SemaphoreType.DMA,
            pltpu.SemaphoreType.DMA,
            pltpu.SemaphoreType.DMA,
        ],
        compiler_params=pltpu.CompilerParams(use_tc_tiling_on_sc=True),
    )(x2d, pe_table)


def _tc_body(x_ref, pe_ref, o_ref):
    o_ref[0] = x_ref[0] + pe_ref[...]


def _tc_add(x, pe_table, k):
    B, S, D = x.shape
    return pl.pallas_call(
        _tc_body,
        grid=(S // _TS, B - k),
        in_specs=[
            pl.BlockSpec((1, _TS, D), lambda i, b: (b + _SC_BATCH, i, 0)),
            pl.BlockSpec((_TS, D), lambda i, b: (i, 0)),
        ],
        out_specs=pl.BlockSpec((1, _TS, D), lambda i, b: (b, i, 0)),
        out_shape=jax.ShapeDtypeStruct((B - k, S, D), x.dtype),
    )(x, pe_table)


def kernel(x, pe_table):
    B, S, D = x.shape
    k = _SC_BATCH
    out_sc = _sc_add(x.reshape(B * S, D), pe_table, k, S).reshape(k, S, D)
    out_tc = _tc_add(x, pe_table, k)
    return jnp.concatenate([out_sc, out_tc], axis=0)


# speedup vs baseline: 1.0841x; 1.0841x over previous
import functools
import jax
import jax.numpy as jnp
from jax import lax
from jax.experimental import pallas as pl
from jax.experimental.pallas import tpu as pltpu, tpu_sc as plsc

_NC, _NS = 2, 16
_NW = _NC * _NS
_L = 16
_CR = 32
_D = 1024
_SC_BATCH = 1
_TS = 512


def _sc_body(x_hbm, pe_hbm, out_hbm, pe_buf, xb0, xb1, l0, l1, s0, s1,
             *, n_batch, seq):
    wid = lax.axis_index("s") * _NC + lax.axis_index("c")
    slabs = seq // (2 * _CR)
    per0 = (wid % slabs) * (2 * _CR)
    xb = (xb0, xb1)
    lsem = (l0, l1)
    ssem = (s0, s1)
    pending_store = [None, None]

    def row0(p, b):
        return b * seq + per0 + p * _CR

    def start_load(p, b, j):
        if pending_store[j] is not None:
            pending_store[j].wait()
            pending_store[j] = None
        return pltpu.async_copy(x_hbm.at[pl.ds(row0(p, b), _CR), :], xb[j],
                                lsem[j])

    chunks = [(p, b) for p in range(2) for b in range(n_batch)]
    load = start_load(*chunks[0], 0)
    for i, (p, b) in enumerate(chunks):
        j = i % 2
        if b == 0:
            pltpu.sync_copy(pe_hbm.at[pl.ds(per0 + p * _CR, _CR), :], pe_buf)
        load.wait()
        if i + 1 < len(chunks):
            load = start_load(*chunks[i + 1], (i + 1) % 2)

        @plsc.parallel_loop(0, _CR, step=1, unroll=1)
        def _(r):
            for c in range(0, _D, _L):
                plsc.addupdate(xb[j].at[r, pl.ds(c, _L)],
                               pe_buf[r, pl.ds(c, _L)])

        pending_store[j] = pltpu.async_copy(
            xb[j], out_hbm.at[pl.ds(row0(p, b), _CR), :], ssem[j])
    for j in range(2):
        if pending_store[j] is not None:
            pending_store[j].wait()


def _sc_add(x2d, pe_table, k, seq):
    mesh = plsc.VectorSubcoreMesh(core_axis_name="c", subcore_axis_name="s",
                                  num_cores=_NC, num_subcores=_NS)
    return pl.kernel(
        functools.partial(_sc_body, n_batch=k, seq=seq),
        out_type=jax.ShapeDtypeStruct(x2d.shape, jnp.float32),
        mesh=mesh,
        scratch_types=[
            pltpu.VMEM((_CR, _D), jnp.float32),
            pltpu.VMEM((_CR, _D), jnp.float32),
            pltpu.VMEM((_CR, _D), jnp.float32),
            pltpu.SemaphoreType.DMA,
            pltpu.SemaphoreType.DMA,
            pltpu.SemaphoreType.DMA,
            pltpu.SemaphoreType.DMA,
        ],
        compiler_params=pltpu.CompilerParams(use_tc_tiling_on_sc=True),
    )(x2d, pe_table)


def _tc_body(x_ref, pe_ref, o_ref):
    o_ref[0] = x_ref[0] + pe_ref[...]


def _tc_add(x, pe_table, k):
    B, S, D = x.shape
    return pl.pallas_call(
        _tc_body,
        grid=(S // _TS, B - k),
        in_specs=[
            pl.BlockSpec((1, _TS, D), lambda i, b: (b + _SC_BATCH, i, 0)),
            pl.BlockSpec((_TS, D), lambda i, b: (i, 0)),
        ],
        out_specs=pl.BlockSpec((1, _TS, D), lambda i, b: (b, i, 0)),
        out_shape=jax.ShapeDtypeStruct((B - k, S, D), x.dtype),
    )(x, pe_table)


def kernel(x, pe_table):
    B, S, D = x.shape
    k = _SC_BATCH
    out_sc = _sc_add(x.reshape(B * S, D), pe_table, k, S).reshape(B, S, D)
    out_tc = _tc_add(x, pe_table, k)
    return jax.lax.dynamic_update_slice(out_sc, out_tc, (k, 0, 0))


# speedup vs baseline: 1.1793x; 1.0879x over previous
import functools
import jax
import jax.numpy as jnp
from jax import lax
from jax.experimental import pallas as pl
from jax.experimental.pallas import tpu as pltpu, tpu_sc as plsc

_NC, _NS = 2, 16
_NW = _NC * _NS
_L = 16
_CR = 32
_D = 1024
_TS = 512


def _sc_body(x_hbm, pe_hbm, out_hbm, pe_buf, xb0, xb1, l0, l1, s0, s1,
             *, sc_batch, seq):
    wid = lax.axis_index("s") * _NC + lax.axis_index("c")
    per0 = wid * (2 * _CR)
    xb = (xb0, xb1)
    lsem = (l0, l1)
    ssem = (s0, s1)
    pending_store = [None, None]

    def start_load(p, j):
        if pending_store[j] is not None:
            pending_store[j].wait()
            pending_store[j] = None
        return pltpu.async_copy(
            x_hbm.at[pl.ds(sc_batch * seq + per0 + p * _CR, _CR), :],
            xb[j], lsem[j])

    load = start_load(0, 0)
    for p in range(2):
        j = p % 2
        pltpu.sync_copy(pe_hbm.at[pl.ds(per0 + p * _CR, _CR), :], pe_buf)
        load.wait()
        if p + 1 < 2:
            load = start_load(p + 1, (p + 1) % 2)

        @plsc.parallel_loop(0, _CR, step=1, unroll=1)
        def _(r):
            for c in range(0, _D, _L):
                plsc.addupdate(xb[j].at[r, pl.ds(c, _L)],
                               pe_buf[r, pl.ds(c, _L)])

        pending_store[j] = pltpu.async_copy(
            xb[j], out_hbm.at[0, pl.ds(per0 + p * _CR, _CR), :], ssem[j])
    for j in range(2):
        if pending_store[j] is not None:
            pending_store[j].wait()


def _sc_add_last_batch(x2d, pe_table, sc_batch, seq):
    mesh = plsc.VectorSubcoreMesh(core_axis_name="c", subcore_axis_name="s",
                                  num_cores=_NC, num_subcores=_NS)
    return pl.kernel(
        functools.partial(_sc_body, sc_batch=sc_batch, seq=seq),
        out_type=jax.ShapeDtypeStruct((1, seq, _D), jnp.float32),
        mesh=mesh,
        scratch_types=[
            pltpu.VMEM((_CR, _D), jnp.float32),
            pltpu.VMEM((_CR, _D), jnp.float32),
            pltpu.VMEM((_CR, _D), jnp.float32),
            pltpu.SemaphoreType.DMA,
            pltpu.SemaphoreType.DMA,
            pltpu.SemaphoreType.DMA,
            pltpu.SemaphoreType.DMA,
        ],
        compiler_params=pltpu.CompilerParams(use_tc_tiling_on_sc=True),
    )(x2d, pe_table)


def _tc_body(x_ref, pe_ref, o_ref):
    o_ref[0] = x_ref[0] + pe_ref[...]


def _tc_add(x, pe_table):
    B, S, D = x.shape
    return pl.pallas_call(
        _tc_body,
        grid=(S // _TS, B),
        in_specs=[
            pl.BlockSpec((1, _TS, D), lambda i, b: (jnp.minimum(b, B - 2), i, 0)),
            pl.BlockSpec((_TS, D), lambda i, b: (i, 0)),
        ],
        out_specs=pl.BlockSpec((1, _TS, D), lambda i, b: (b, i, 0)),
        out_shape=jax.ShapeDtypeStruct((B, S, D), x.dtype),
    )(x, pe_table)


def _patch_body(full_ref, sc_ref, o_ref):
    o_ref[...] = sc_ref[...]


def _patch_last_batch(out_tc, out_sc):
    B, S, D = out_tc.shape
    return pl.pallas_call(
        _patch_body,
        grid=(S // _TS,),
        in_specs=[
            pl.BlockSpec(memory_space=pl.ANY),
            pl.BlockSpec((1, _TS, D), lambda i: (0, i, 0)),
        ],
        out_specs=pl.BlockSpec((1, _TS, D), lambda i: (B - 1, i, 0)),
        out_shape=jax.ShapeDtypeStruct((B, S, D), out_tc.dtype),
        input_output_aliases={0: 0},
    )(out_tc, out_sc)


def kernel(x, pe_table):
    B, S, D = x.shape
    out_sc = _sc_add_last_batch(x.reshape(B * S, D), pe_table, B - 1, S)
    out_tc = _tc_add(x, pe_table)
    return _patch_last_batch(out_tc, out_sc)


# speedup vs baseline: 1.3045x; 1.1062x over previous
import functools
import jax
import jax.numpy as jnp
from jax import lax
from jax.experimental import pallas as pl
from jax.experimental.pallas import tpu as pltpu, tpu_sc as plsc

_NC, _NS = 2, 16
_NW = _NC * _NS
_L = 16
_CR = 32
_D = 1024
_TS = 512


def _sc_body(x_hbm, pe_hbm, out_hbm, pe_buf, xb0, xb1, l0, l1, s0, s1,
             *, sc_batch, seq):
    wid = lax.axis_index("s") * _NC + lax.axis_index("c")
    per0 = wid * (2 * _CR)
    xb = (xb0, xb1)
    lsem = (l0, l1)
    ssem = (s0, s1)
    pending_store = [None, None]

    def start_load(p, j):
        if pending_store[j] is not None:
            pending_store[j].wait()
            pending_store[j] = None
        return pltpu.async_copy(
            x_hbm.at[pl.ds(sc_batch * seq + per0 + p * _CR, _CR), :],
            xb[j], lsem[j])

    load = start_load(0, 0)
    for p in range(2):
        j = p % 2
        pltpu.sync_copy(pe_hbm.at[pl.ds(per0 + p * _CR, _CR), :], pe_buf)
        load.wait()
        if p + 1 < 2:
            load = start_load(p + 1, (p + 1) % 2)

        @plsc.parallel_loop(0, _CR, step=1, unroll=1)
        def _(r):
            for c in range(0, _D, _L):
                plsc.addupdate(xb[j].at[r, pl.ds(c, _L)],
                               pe_buf[r, pl.ds(c, _L)])

        pending_store[j] = pltpu.async_copy(
            xb[j], out_hbm.at[0, pl.ds(per0 + p * _CR, _CR), :], ssem[j])
    for j in range(2):
        if pending_store[j] is not None:
            pending_store[j].wait()


def _sc_add_last_batch(x2d, pe_table, sc_batch, seq):
    mesh = plsc.VectorSubcoreMesh(core_axis_name="c", subcore_axis_name="s",
                                  num_cores=_NC, num_subcores=_NS)
    return pl.kernel(
        functools.partial(_sc_body, sc_batch=sc_batch, seq=seq),
        out_type=jax.ShapeDtypeStruct((1, seq, _D), jnp.float32),
        mesh=mesh,
        scratch_types=[
            pltpu.VMEM((_CR, _D), jnp.float32),
            pltpu.VMEM((_CR, _D), jnp.float32),
            pltpu.VMEM((_CR, _D), jnp.float32),
            pltpu.SemaphoreType.DMA,
            pltpu.SemaphoreType.DMA,
            pltpu.SemaphoreType.DMA,
            pltpu.SemaphoreType.DMA,
        ],
        compiler_params=pltpu.CompilerParams(use_tc_tiling_on_sc=True),
    )(x2d, pe_table)


def _tc_body(x_ref, pe_ref, o_ref):
    o_ref[0] = x_ref[0] + pe_ref[...]


def _tc_add(x, pe_table):
    B, S, D = x.shape
    return pl.pallas_call(
        _tc_body,
        grid=(S // _TS, B - 1),
        in_specs=[
            pl.BlockSpec((1, _TS, D), lambda i, b: (b, i, 0)),
            pl.BlockSpec((_TS, D), lambda i, b: (i, 0)),
        ],
        out_specs=pl.BlockSpec((1, _TS, D), lambda i, b: (b, i, 0)),
        out_shape=jax.ShapeDtypeStruct((B, S, D), x.dtype),
    )(x, pe_table)


def _patch_body(full_ref, sc_ref, o_ref):
    o_ref[...] = sc_ref[...]


def _patch_last_batch(out_tc, out_sc):
    B, S, D = out_tc.shape
    return pl.pallas_call(
        _patch_body,
        grid=(S // _TS,),
        in_specs=[
            pl.BlockSpec(memory_space=pl.ANY),
            pl.BlockSpec((1, _TS, D), lambda i: (0, i, 0)),
        ],
        out_specs=pl.BlockSpec((1, _TS, D), lambda i: (B - 1, i, 0)),
        out_shape=jax.ShapeDtypeStruct((B, S, D), out_tc.dtype),
        input_output_aliases={0: 0},
    )(out_tc, out_sc)


def kernel(x, pe_table):
    B, S, D = x.shape
    out_sc = _sc_add_last_batch(x.reshape(B * S, D), pe_table, B - 1, S)
    out_tc = _tc_add(x, pe_table)
    return _patch_last_batch(out_tc, out_sc)
